# baseline (device time: 98703 ns/iter reference)
import jax
import jax.numpy as jnp
from jax import lax
from jax.experimental import pallas as pl
from jax.experimental.pallas import tpu as pltpu

M = 1024
N = 1024


def kernel(A, B):
    def body(a_ref, b_ref, out_ref, recv_ref, send_sems, recv_sems):
        p = lax.axis_index("i")
        kb1 = (p // 2) % 2
        kb2 = p % 2
        kb3 = (p // 4) % 2

        barrier = pltpu.get_barrier_semaphore()
        for mask in (3, 1, 4):
            pl.semaphore_signal(
                barrier,
                inc=1,
                device_id=(lax.bitwise_xor(p, mask),),
                device_id_type=pl.DeviceIdType.MESH,
            )
        pl.semaphore_wait(barrier, 3)

        out_ref[:, :] = jnp.dot(
            a_ref[:, :].astype(jnp.bfloat16),
            b_ref[:, :].astype(jnp.bfloat16),
            preferred_element_type=jnp.float32,
        )

        keep1 = kb1 * 512
        send1 = (1 - kb1) * 512
        keep2 = keep1 + kb2 * 256
        send2 = keep1 + (1 - kb2) * 256
        keep3 = keep2 + kb3 * 128
        send3 = keep2 + (1 - kb3) * 128

        rs_stages = [
            (3, send1, keep1, 0, 512),
            (1, send2, keep2, 512, 256),
            (4, send3, keep3, 768, 128),
        ]
        for s, (mask, sbase, kbase, roff, size) in enumerate(rs_stages):
            partner = lax.bitwise_xor(p, mask)
            rdma = pltpu.make_async_remote_copy(
                src_ref=out_ref.at[pl.ds(sbase, size), :],
                dst_ref=recv_ref.at[pl.ds(roff, size), :],
                send_sem=send_sems.at[s],
                recv_sem=recv_sems.at[s],
                device_id=(partner,),
                device_id_type=pl.DeviceIdType.MESH,
            )
            rdma.start()
            rdma.wait()
            out_ref[pl.ds(kbase, size), :] = (
                out_ref[pl.ds(kbase, size), :]
                + recv_ref[pl.ds(roff, size), :]
            )

        out_ref[pl.ds(keep3, 128), :] = jnp.maximum(
            out_ref[pl.ds(keep3, 128), :], 0.0
        )

        ag_stages = [
            (4, keep3, 128, 3),
            (1, keep2, 256, 4),
            (3, keep1, 512, 5),
        ]
        for mask, sbase, size, s in ag_stages:
            partner = lax.bitwise_xor(p, mask)
            rdma = pltpu.make_async_remote_copy(
                src_ref=out_ref.at[pl.ds(sbase, size), :],
                dst_ref=out_ref.at[pl.ds(sbase, size), :],
                send_sem=send_sems.at[s],
                recv_sem=recv_sems.at[s],
                device_id=(partner,),
                device_id_type=pl.DeviceIdType.MESH,
            )
            rdma.start()
            rdma.wait()

    return pl.pallas_call(
        body,
        out_shape=jax.ShapeDtypeStruct((M, N), jnp.float32),
        in_specs=[
            pl.BlockSpec(memory_space=pltpu.VMEM),
            pl.BlockSpec(memory_space=pltpu.VMEM),
        ],
        out_specs=pl.BlockSpec(memory_space=pltpu.VMEM),
        scratch_shapes=[
            pltpu.VMEM((896, N), jnp.float32),
            pltpu.SemaphoreType.DMA((6,)),
            pltpu.SemaphoreType.DMA((6,)),
        ],
        compiler_params=pltpu.CompilerParams(collective_id=0),
    )(A, B)


# device time: 59154 ns/iter; 1.6686x vs baseline; 1.6686x over previous
import jax
import jax.numpy as jnp
from jax import lax
from jax.experimental import pallas as pl
from jax.experimental.pallas import tpu as pltpu

M = 1024
N = 1024


def kernel(A, B):
    def body(a_ref, b_ref, out_ref, acc_ref, sbuf_ref, rbuf_ref, send_sems, recv_sems):
        p = lax.axis_index("i")
        kb1 = (p // 2) % 2
        kb2 = p % 2
        kb3 = (p // 4) % 2

        barrier = pltpu.get_barrier_semaphore()
        for mask in (3, 1, 4):
            pl.semaphore_signal(
                barrier,
                inc=1,
                device_id=(lax.bitwise_xor(p, mask),),
                device_id_type=pl.DeviceIdType.MESH,
            )
        pl.semaphore_wait(barrier, 3)

        acc_ref[:, :] = jnp.dot(
            a_ref[:, :].astype(jnp.bfloat16),
            b_ref[:, :].astype(jnp.bfloat16),
            preferred_element_type=jnp.float32,
        )

        keep1 = kb1 * 512
        send1 = (1 - kb1) * 512
        keep2 = keep1 + kb2 * 256
        send2 = keep1 + (1 - kb2) * 256
        keep3 = keep2 + kb3 * 128
        send3 = keep2 + (1 - kb3) * 128

        rs_stages = [
            (3, send1, keep1, 0, 512),
            (1, send2, keep2, 512, 256),
            (4, send3, keep3, 768, 128),
        ]
        for s, (mask, sbase, kbase, off, size) in enumerate(rs_stages):
            partner = lax.bitwise_xor(p, mask)
            sbuf_ref[pl.ds(off, size), :] = acc_ref[
                pl.ds(sbase, size), :
            ].astype(jnp.bfloat16)
            rdma = pltpu.make_async_remote_copy(
                src_ref=sbuf_ref.at[pl.ds(off, size), :],
                dst_ref=rbuf_ref.at[pl.ds(off, size), :],
                send_sem=send_sems.at[s],
                recv_sem=recv_sems.at[s],
                device_id=(partner,),
                device_id_type=pl.DeviceIdType.MESH,
            )
            rdma.start()
            rdma.wait()
            acc_ref[pl.ds(kbase, size), :] = (
                acc_ref[pl.ds(kbase, size), :]
                + rbuf_ref[pl.ds(off, size), :].astype(jnp.float32)
            )

        out_ref[pl.ds(keep3, 128), :] = jnp.maximum(
            acc_ref[pl.ds(keep3, 128), :], 0.0
        ).astype(jnp.bfloat16)

        ag_stages = [
            (4, keep3, 128, 3),
            (1, keep2, 256, 4),
            (3, keep1, 512, 5),
        ]
        for mask, sbase, size, s in ag_stages:
            partner = lax.bitwise_xor(p, mask)
            rdma = pltpu.make_async_remote_copy(
                src_ref=out_ref.at[pl.ds(sbase, size), :],
                dst_ref=out_ref.at[pl.ds(sbase, size), :],
                send_sem=send_sems.at[s],
                recv_sem=recv_sems.at[s],
                device_id=(partner,),
                device_id_type=pl.DeviceIdType.MESH,
            )
            rdma.start()
            rdma.wait()

    return pl.pallas_call(
        body,
        out_shape=jax.ShapeDtypeStruct((M, N), jnp.bfloat16),
        in_specs=[
            pl.BlockSpec(memory_space=pltpu.VMEM),
            pl.BlockSpec(memory_space=pltpu.VMEM),
        ],
        out_specs=pl.BlockSpec(memory_space=pltpu.VMEM),
        scratch_shapes=[
            pltpu.VMEM((M, N), jnp.float32),
            pltpu.VMEM((896, N), jnp.bfloat16),
            pltpu.VMEM((896, N), jnp.bfloat16),
            pltpu.SemaphoreType.DMA((6,)),
            pltpu.SemaphoreType.DMA((6,)),
        ],
        compiler_params=pltpu.CompilerParams(collective_id=0),
    )(A, B)


# device time: 39705 ns/iter; 2.4859x vs baseline; 1.4898x over previous
import jax
import jax.numpy as jnp
from jax import lax
from jax.experimental import pallas as pl
from jax.experimental.pallas import tpu as pltpu

M = 1024
N = 1024
SIZES = (256, 128, 64)


def kernel(A, B):
    def body(a_ref, b_ref, out_ref, acc_ref, sbuf_ref, rbuf_ref,
             send_sems, recv_sems):
        p = lax.axis_index("i")
        bit0 = p % 2
        bit1 = (p // 2) % 2
        bit2 = (p // 4) % 2

        barrier = pltpu.get_barrier_semaphore()
        for mask in (3, 1, 4):
            pl.semaphore_signal(
                barrier,
                inc=1,
                device_id=(lax.bitwise_xor(p, mask),),
                device_id_type=pl.DeviceIdType.MESH,
            )
        pl.semaphore_wait(barrier, 3)

        parts = []
        for base, masks, bits, soff, sem0 in (
            (0, (3, 1, 4), (bit1, bit0, bit2), (0, 256, 384), 0),
            (512, (1, 4, 3), (bit0, bit2, bit1), (448, 704, 832), 6),
        ):
            keep, send = [], []
            cur = base
            for t in range(3):
                sz, b = SIZES[t], bits[t]
                keep.append(cur + b * sz)
                send.append(cur + (1 - b) * sz)
                cur = keep[t]
            parts.append(dict(
                masks=masks, keep=keep, send=send, soff=soff, sem0=sem0,
            ))

        inflight = {}

        def rs_start(pi, t):
            pt = parts[pi]
            sz, off = SIZES[t], pt["soff"][t]
            sbuf_ref[pl.ds(off, sz), :] = acc_ref[
                pl.ds(pt["send"][t], sz), :
            ].astype(jnp.bfloat16)
            rdma = pltpu.make_async_remote_copy(
                src_ref=sbuf_ref.at[pl.ds(off, sz), :],
                dst_ref=rbuf_ref.at[pl.ds(off, sz), :],
                send_sem=send_sems.at[pt["sem0"] + t],
                recv_sem=recv_sems.at[pt["sem0"] + t],
                device_id=(lax.bitwise_xor(p, pt["masks"][t]),),
                device_id_type=pl.DeviceIdType.MESH,
            )
            rdma.start()
            inflight[("rs", pi, t)] = rdma

        def rs_finish(pi, t):
            pt = parts[pi]
            sz, off = SIZES[t], pt["soff"][t]
            inflight.pop(("rs", pi, t)).wait()
            acc_ref[pl.ds(pt["keep"][t], sz), :] = (
                acc_ref[pl.ds(pt["keep"][t], sz), :]
                + rbuf_ref[pl.ds(off, sz), :].astype(jnp.float32)
            )

        def ag_start(pi, t):
            pt = parts[pi]
            sz, sbase = SIZES[2 - t], pt["keep"][2 - t]
            rdma = pltpu.make_async_remote_copy(
                src_ref=out_ref.at[pl.ds(sbase, sz), :],
                dst_ref=out_ref.at[pl.ds(sbase, sz), :],
                send_sem=send_sems.at[pt["sem0"] + 3 + t],
                recv_sem=recv_sems.at[pt["sem0"] + 3 + t],
                device_id=(lax.bitwise_xor(p, pt["masks"][2 - t]),),
                device_id_type=pl.DeviceIdType.MESH,
            )
            rdma.start()
            inflight[("ag", pi, t)] = rdma

        acc_ref[0:512, :] = jnp.dot(
            a_ref[0:512, :].astype(jnp.bfloat16),
            b_ref[:, :].astype(jnp.bfloat16),
            preferred_element_type=jnp.float32,
        )
        rs_start(0, 0)
        acc_ref[512:M, :] = jnp.dot(
            a_ref[512:M, :].astype(jnp.bfloat16),
            b_ref[:, :].astype(jnp.bfloat16),
            preferred_element_type=jnp.float32,
        )
        rs_start(1, 0)

        for t in range(3):
            for pi in (0, 1):
                rs_finish(pi, t)
                if t < 2:
                    rs_start(pi, t + 1)
                else:
                    k3 = parts[pi]["keep"][2]
                    out_ref[pl.ds(k3, 64), :] = jnp.maximum(
                        acc_ref[pl.ds(k3, 64), :], 0.0
                    ).astype(jnp.bfloat16)
                    ag_start(pi, 0)

        for t in range(3):
            for pi in (0, 1):
                inflight.pop(("ag", pi, t)).wait()
                if t < 2:
                    ag_start(pi, t + 1)

    return pl.pallas_call(
        body,
        out_shape=jax.ShapeDtypeStruct((M, N), jnp.bfloat16),
        in_specs=[
            pl.BlockSpec(memory_space=pltpu.VMEM),
            pl.BlockSpec(memory_space=pltpu.VMEM),
        ],
        out_specs=pl.BlockSpec(memory_space=pltpu.VMEM),
        scratch_shapes=[
            pltpu.VMEM((M, N), jnp.float32),
            pltpu.VMEM((896, N), jnp.bfloat16),
            pltpu.VMEM((896, N), jnp.bfloat16),
            pltpu.SemaphoreType.DMA((12,)),
            pltpu.SemaphoreType.DMA((12,)),
        ],
        compiler_params=pltpu.CompilerParams(collective_id=0),
    )(A, B)
